# baseline (speedup 1.0000x reference)
import functools

import jax
import jax.numpy as jnp
from jax import lax
from jax.experimental import pallas as pl
from jax.experimental.pallas import tpu as pltpu
from jax.experimental.pallas import tpu_sc as plsc

_B, _T, _D, _M = 16, 512, 384, 4096
_EPS = 1e-5
_L = 16
_CH = 128
_NCH = _M // _CH // 2
_ZROW = _B * _T


_BB = 4


def _predictor_body(x_ref, w1_ref, b1_ref, g1_ref, be1_ref,
                    w2_ref, b2_ref, g2_ref, be2_ref, wo_ref, bo_ref, out_ref):
    def conv(h, w_ref, b):
        h = h.astype(jnp.bfloat16)
        z = jnp.zeros((1, _D), jnp.bfloat16)
        hl = jnp.concatenate([z, h[:-1]], axis=0)
        hr = jnp.concatenate([h[1:], z], axis=0)
        y = (jnp.dot(hl, w_ref[0], preferred_element_type=jnp.float32)
             + jnp.dot(h, w_ref[1], preferred_element_type=jnp.float32)
             + jnp.dot(hr, w_ref[2], preferred_element_type=jnp.float32))
        return y + b

    def ln(h, g, b):
        m = jnp.mean(h, axis=-1, keepdims=True)
        c = h - m
        v = jnp.mean(c * c, axis=-1, keepdims=True)
        return c * lax.rsqrt(v + _EPS) * g + b

    for i in range(_BB):
        h = x_ref[i]
        h = ln(jax.nn.relu(conv(h, w1_ref, b1_ref[0])), g1_ref[0], be1_ref[0])
        h = ln(jax.nn.relu(conv(h, w2_ref, b2_ref[0])), g2_ref[0], be2_ref[0])
        lp = jnp.dot(h, wo_ref[...], preferred_element_type=jnp.float32) + bo_ref[0, 0]
        out_ref[i, 0] = lp[:, 0]


def _regulate_body(x_hbm, dur_hbm, zrows_hbm, zidx_hbm, out_hbm,
                   dur_v, cum_v, idx_v, gbuf0, gbuf1, zbuf,
                   sem_g0, sem_g1, sem_w0, sem_w1, sem_z):
    cid = lax.axis_index("c")
    sid = lax.axis_index("s")
    b = cid * 8 + sid // 2
    half = sid % 2

    pltpu.sync_copy(dur_hbm.at[b], dur_v)

    @pl.when(sid == 0)
    def _():
        pltpu.sync_copy(zrows_hbm, zbuf)

    lane = lax.iota(jnp.int32, _L)
    _full15 = jnp.full((_L,), _L - 1, jnp.int32)
    _dn = lax.GatherDimensionNumbers(offset_dims=(), collapsed_slice_dims=(0,),
                                     start_index_map=(0,))

    def bcast_last(v):
        return lax.gather(v, _full15[:, None], _dn, (1,),
                          mode=lax.GatherScatterMode.PROMISE_IN_BOUNDS)

    def cum_step(t, carry):
        dv = dur_v[pl.ds(t * _L, _L)]
        cs = plsc.cumsum(dv) + carry
        cum_v[pl.ds(t * _L, _L)] = cs
        return bcast_last(cs)

    total_v = lax.fori_loop(0, _T // _L, cum_step,
                            jnp.zeros((_L,), jnp.int32), unroll=False)
    total = jnp.max(total_v)

    nvc = (total + _CH - 1) // _CH
    nv = jnp.maximum((nvc - half + 1) // 2, 0)

    plsc.subcore_barrier()

    def zfire(k, _):
        c = 2 * k + half
        pltpu.async_copy(zbuf, out_hbm.at[b, pl.ds(c * _CH, _CH)], sem_z)
        return 0

    lax.fori_loop(nv, _NCH, zfire, 0, unroll=False)

    pltpu.sync_copy(zidx_hbm, idx_v)

    def scat_step(t, _):
        dv = dur_v[pl.ds(t * _L, _L)]
        cs = cum_v[pl.ds(t * _L, _L)]
        starts = cs - dv
        gi = lane + t * _L
        plsc.store_scatter(idx_v, [jnp.clip(starts, 0, _M - 1)], gi + 1,
                           mask=dv > 0)
        return 0

    lax.fori_loop(0, _T // _L, scat_step, 0, unroll=False)

    def idx_step(i, carry):
        r16 = idx_v[pl.ds(i * _L, _L)]
        cmc = jnp.maximum(plsc.cummax(r16), carry)
        idx_v[pl.ds(i * _L, _L)] = cmc - 1 + b * _T
        return bcast_last(cmc)

    lax.fori_loop(0, nvc * (_CH // _L), idx_step, jnp.zeros((_L,), jnp.int32),
                  unroll=False)

    bufs = ((gbuf0, sem_g0, sem_w0), (gbuf1, sem_g1, sem_w1))

    def issue_gather(k, gbuf, sem_gp):
        off = (2 * k + half) * _CH
        pltpu.async_copy(x_hbm.at[idx_v.at[pl.ds(off, _CH)]], gbuf, sem_gp)

    @pl.when(nv >= 1)
    def _():
        issue_gather(0, gbuf0, sem_g0)

    def gather_step(k, _):
        off = (2 * k + half) * _CH
        rem = total - off

        def run(cur, nxt):
            gbuf, sem_gp, sem_w = cur
            obuf, sem_go, sem_wo = nxt

            @pl.when(k + 1 < nv)
            def _():
                @pl.when(k >= 1)
                def _():
                    pltpu.make_async_copy(x_hbm.at[pl.ds(0, _CH)], obuf, sem_wo).wait()

                issue_gather(k + 1, obuf, sem_go)

            pltpu.make_async_copy(x_hbm.at[pl.ds(0, _CH)], gbuf, sem_gp).wait()

            @pl.when(rem < _CH)
            def _():
                zf = jnp.zeros((_L,), jnp.float32)

                def zrow(r, _):
                    for j in range(_D // _L):
                        gbuf[r, pl.ds(j * _L, _L)] = zf
                    return 0

                lax.fori_loop(rem, _CH, zrow, 0, unroll=False)

            pltpu.async_copy(gbuf, out_hbm.at[b, pl.ds(off, _CH)], sem_w)

        for r in range(2):
            @pl.when(k % 2 == r)
            def _(r=r):
                run(bufs[r], bufs[(r + 1) % 2])

        return 0

    lax.fori_loop(0, nv, gather_step, 0, unroll=False)

    for r in range(2):
        @pl.when(jnp.minimum(nv, 2) > r)
        def _(r=r):
            pltpu.make_async_copy(x_hbm.at[pl.ds(0, _CH)], bufs[r][0], bufs[r][2]).wait()

    def zdrain(k, _):
        pltpu.make_async_copy(x_hbm.at[pl.ds(0, _CH)], zbuf, sem_z).wait()
        return 0

    lax.fori_loop(nv, _NCH, zdrain, 0, unroll=False)


def kernel(x, teacher_durations, mel_spec_length, W1, b1, g1, be1,
           W2, b2, g2, be2, Wo, bo):
    x = x.astype(jnp.float32)

    log_pred = pl.pallas_call(
        _predictor_body,
        grid=(_B // _BB,),
        in_specs=[
            pl.BlockSpec((_BB, _T, _D), lambda i: (i, 0, 0)),
            pl.BlockSpec((3, _D, _D), lambda i: (0, 0, 0)),
            pl.BlockSpec((1, _D), lambda i: (0, 0)),
            pl.BlockSpec((1, _D), lambda i: (0, 0)),
            pl.BlockSpec((1, _D), lambda i: (0, 0)),
            pl.BlockSpec((3, _D, _D), lambda i: (0, 0, 0)),
            pl.BlockSpec((1, _D), lambda i: (0, 0)),
            pl.BlockSpec((1, _D), lambda i: (0, 0)),
            pl.BlockSpec((1, _D), lambda i: (0, 0)),
            pl.BlockSpec((_D, 1), lambda i: (0, 0)),
            pl.BlockSpec((1, 1), lambda i: (0, 0)),
        ],
        out_specs=pl.BlockSpec((_BB, 1, _T), lambda i: (i, 0, 0)),
        out_shape=jax.ShapeDtypeStruct((_B, 1, _T), jnp.float32),
    )(
        x,
        jnp.transpose(W1, (2, 1, 0)).astype(jnp.bfloat16),
        b1.reshape(1, _D), g1.reshape(1, _D), be1.reshape(1, _D),
        jnp.transpose(W2, (2, 1, 0)).astype(jnp.bfloat16),
        b2.reshape(1, _D), g2.reshape(1, _D), be2.reshape(1, _D),
        Wo.astype(jnp.float32),
        bo.reshape(1, 1).astype(jnp.float32),
    )[:, 0, :]

    table = x.reshape(_B * _T, _D)
    dur = teacher_durations.astype(jnp.int32)

    regulate = functools.partial(
        pl.kernel,
        out_type=jax.ShapeDtypeStruct((_B, _M, _D), jnp.float32),
        mesh=plsc.VectorSubcoreMesh(core_axis_name="c", subcore_axis_name="s"),
        scratch_types=[
            pltpu.VMEM((_T,), jnp.int32),
            pltpu.VMEM((_T,), jnp.int32),
            pltpu.VMEM((_M,), jnp.int32),
            pltpu.VMEM((_CH, _D), jnp.float32),
            pltpu.VMEM((_CH, _D), jnp.float32),
            pltpu.VMEM_SHARED((_CH, _D), jnp.float32),
            pltpu.SemaphoreType.DMA,
            pltpu.SemaphoreType.DMA,
            pltpu.SemaphoreType.DMA,
            pltpu.SemaphoreType.DMA,
            pltpu.SemaphoreType.DMA,
        ],
        compiler_params=pltpu.CompilerParams(needs_layout_passes=False),
    )(_regulate_body)

    x_out = regulate(table, dur, jnp.zeros((_CH, _D), jnp.float32),
                     jnp.zeros((_M,), jnp.int32))
    return (x_out, log_pred)


# speedup vs baseline: 1.0768x; 1.0768x over previous
import functools

import jax
import jax.numpy as jnp
from jax import lax
from jax.experimental import pallas as pl
from jax.experimental.pallas import tpu as pltpu
from jax.experimental.pallas import tpu_sc as plsc

_B, _T, _D, _M = 16, 512, 384, 4096
_EPS = 1e-5
_L = 16
_CH = 64
_NCH = _M // _CH // 2
_ZROW = _B * _T


_BB = 2


def _predictor_body(x_ref, w1_ref, b1_ref, g1_ref, be1_ref,
                    w2_ref, b2_ref, g2_ref, be2_ref, wo_ref, bo_ref, out_ref):
    def conv(h, w_ref, b):
        h = h.astype(jnp.bfloat16)
        z = jnp.zeros((1, _D), jnp.bfloat16)
        hl = jnp.concatenate([z, h[:-1]], axis=0)
        hr = jnp.concatenate([h[1:], z], axis=0)
        y = (jnp.dot(hl, w_ref[0], preferred_element_type=jnp.float32)
             + jnp.dot(h, w_ref[1], preferred_element_type=jnp.float32)
             + jnp.dot(hr, w_ref[2], preferred_element_type=jnp.float32))
        return y + b

    def ln(h, g, b):
        m = jnp.mean(h, axis=-1, keepdims=True)
        c = h - m
        v = jnp.mean(c * c, axis=-1, keepdims=True)
        return c * lax.rsqrt(v + _EPS) * g + b

    for i in range(_BB):
        h = x_ref[i]
        h = ln(jax.nn.relu(conv(h, w1_ref, b1_ref[0])), g1_ref[0], be1_ref[0])
        h = ln(jax.nn.relu(conv(h, w2_ref, b2_ref[0])), g2_ref[0], be2_ref[0])
        lp = jnp.dot(h, wo_ref[...], preferred_element_type=jnp.float32) + bo_ref[0, 0]
        out_ref[i, 0] = lp[:, 0]


def _regulate_body(x_hbm, dur_hbm, zrows_hbm, zidx_hbm, out_hbm,
                   dur_v, cum_v, idx_v, gbuf0, gbuf1, gbuf2, zbuf,
                   sem_g0, sem_g1, sem_g2, sem_w0, sem_w1, sem_w2, sem_z):
    cid = lax.axis_index("c")
    sid = lax.axis_index("s")
    b = cid * 8 + sid // 2
    half = sid % 2

    pltpu.sync_copy(dur_hbm.at[b], dur_v)

    @pl.when(sid == 0)
    def _():
        pltpu.sync_copy(zrows_hbm, zbuf)

    lane = lax.iota(jnp.int32, _L)
    _full15 = jnp.full((_L,), _L - 1, jnp.int32)
    _dn = lax.GatherDimensionNumbers(offset_dims=(), collapsed_slice_dims=(0,),
                                     start_index_map=(0,))

    def bcast_last(v):
        return lax.gather(v, _full15[:, None], _dn, (1,),
                          mode=lax.GatherScatterMode.PROMISE_IN_BOUNDS)

    def cum_step(t, carry):
        dv = dur_v[pl.ds(t * _L, _L)]
        cs = plsc.cumsum(dv) + carry
        cum_v[pl.ds(t * _L, _L)] = cs
        return bcast_last(cs)

    total_v = lax.fori_loop(0, _T // _L, cum_step,
                            jnp.zeros((_L,), jnp.int32), unroll=False)
    total = jnp.max(total_v)

    nvc = (total + _CH - 1) // _CH
    nv = jnp.maximum((nvc - half + 1) // 2, 0)

    plsc.subcore_barrier()

    def zfire(k, _):
        c = 2 * k + half
        pltpu.async_copy(zbuf, out_hbm.at[b, pl.ds(c * _CH, _CH)], sem_z)
        return 0

    lax.fori_loop(nv, _NCH, zfire, 0, unroll=False)

    pltpu.sync_copy(zidx_hbm, idx_v)

    def scat_step(t, _):
        dv = dur_v[pl.ds(t * _L, _L)]
        cs = cum_v[pl.ds(t * _L, _L)]
        starts = cs - dv
        gi = lane + t * _L
        plsc.store_scatter(idx_v, [jnp.clip(starts, 0, _M - 1)], gi + 1,
                           mask=dv > 0)
        return 0

    lax.fori_loop(0, _T // _L, scat_step, 0, unroll=False)

    def idx_step(i, carry):
        r16 = idx_v[pl.ds(i * _L, _L)]
        cmc = jnp.maximum(plsc.cummax(r16), carry)
        idx_v[pl.ds(i * _L, _L)] = cmc - 1 + b * _T
        return bcast_last(cmc)

    lax.fori_loop(0, nvc * (_CH // _L), idx_step, jnp.zeros((_L,), jnp.int32),
                  unroll=False)

    bufs = ((gbuf0, sem_g0, sem_w0), (gbuf1, sem_g1, sem_w1),
            (gbuf2, sem_g2, sem_w2))

    def issue_gather(k, gbuf, sem_gp):
        off = (2 * k + half) * _CH
        pltpu.async_copy(x_hbm.at[idx_v.at[pl.ds(off, _CH)]], gbuf, sem_gp)

    @pl.when(nv >= 1)
    def _():
        issue_gather(0, gbuf0, sem_g0)

    @pl.when(nv >= 2)
    def _():
        issue_gather(1, gbuf1, sem_g1)

    def gather_step(k, _):
        off = (2 * k + half) * _CH
        rem = total - off

        def run(cur, nxt):
            gbuf, sem_gp, sem_w = cur
            obuf, sem_go, sem_wo = nxt

            @pl.when(k + 2 < nv)
            def _():
                @pl.when(k >= 1)
                def _():
                    pltpu.make_async_copy(x_hbm.at[pl.ds(0, _CH)], obuf, sem_wo).wait()

                issue_gather(k + 2, obuf, sem_go)

            pltpu.make_async_copy(x_hbm.at[pl.ds(0, _CH)], gbuf, sem_gp).wait()

            @pl.when(rem < _CH)
            def _():
                zf = jnp.zeros((_L,), jnp.float32)

                def zrow(r, _):
                    for j in range(_D // _L):
                        gbuf[r, pl.ds(j * _L, _L)] = zf
                    return 0

                lax.fori_loop(rem, _CH, zrow, 0, unroll=False)

            pltpu.async_copy(gbuf, out_hbm.at[b, pl.ds(off, _CH)], sem_w)

        for r in range(3):
            @pl.when(k % 3 == r)
            def _(r=r):
                run(bufs[r], bufs[(r + 2) % 3])

        return 0

    lax.fori_loop(0, nv, gather_step, 0, unroll=False)

    for r in range(3):
        @pl.when(jnp.minimum(nv, 3) > r)
        def _(r=r):
            pltpu.make_async_copy(x_hbm.at[pl.ds(0, _CH)], bufs[r][0], bufs[r][2]).wait()

    def zdrain(k, _):
        pltpu.make_async_copy(x_hbm.at[pl.ds(0, _CH)], zbuf, sem_z).wait()
        return 0

    lax.fori_loop(nv, _NCH, zdrain, 0, unroll=False)


def kernel(x, teacher_durations, mel_spec_length, W1, b1, g1, be1,
           W2, b2, g2, be2, Wo, bo):
    x = x.astype(jnp.float32)

    log_pred = pl.pallas_call(
        _predictor_body,
        grid=(_B // _BB,),
        in_specs=[
            pl.BlockSpec((_BB, _T, _D), lambda i: (i, 0, 0)),
            pl.BlockSpec((3, _D, _D), lambda i: (0, 0, 0)),
            pl.BlockSpec((1, _D), lambda i: (0, 0)),
            pl.BlockSpec((1, _D), lambda i: (0, 0)),
            pl.BlockSpec((1, _D), lambda i: (0, 0)),
            pl.BlockSpec((3, _D, _D), lambda i: (0, 0, 0)),
            pl.BlockSpec((1, _D), lambda i: (0, 0)),
            pl.BlockSpec((1, _D), lambda i: (0, 0)),
            pl.BlockSpec((1, _D), lambda i: (0, 0)),
            pl.BlockSpec((_D, 1), lambda i: (0, 0)),
            pl.BlockSpec((1, 1), lambda i: (0, 0)),
        ],
        out_specs=pl.BlockSpec((_BB, 1, _T), lambda i: (i, 0, 0)),
        out_shape=jax.ShapeDtypeStruct((_B, 1, _T), jnp.float32),
    )(
        x,
        jnp.transpose(W1, (2, 1, 0)).astype(jnp.bfloat16),
        b1.reshape(1, _D), g1.reshape(1, _D), be1.reshape(1, _D),
        jnp.transpose(W2, (2, 1, 0)).astype(jnp.bfloat16),
        b2.reshape(1, _D), g2.reshape(1, _D), be2.reshape(1, _D),
        Wo.astype(jnp.float32),
        bo.reshape(1, 1).astype(jnp.float32),
    )[:, 0, :]

    table = x.reshape(_B * _T, _D)
    dur = teacher_durations.astype(jnp.int32)

    regulate = functools.partial(
        pl.kernel,
        out_type=jax.ShapeDtypeStruct((_B, _M, _D), jnp.float32),
        mesh=plsc.VectorSubcoreMesh(core_axis_name="c", subcore_axis_name="s"),
        scratch_types=[
            pltpu.VMEM((_T,), jnp.int32),
            pltpu.VMEM((_T,), jnp.int32),
            pltpu.VMEM((_M,), jnp.int32),
            pltpu.VMEM((_CH, _D), jnp.float32),
            pltpu.VMEM((_CH, _D), jnp.float32),
            pltpu.VMEM((_CH, _D), jnp.float32),
            pltpu.VMEM_SHARED((_CH, _D), jnp.float32),
            pltpu.SemaphoreType.DMA,
            pltpu.SemaphoreType.DMA,
            pltpu.SemaphoreType.DMA,
            pltpu.SemaphoreType.DMA,
            pltpu.SemaphoreType.DMA,
            pltpu.SemaphoreType.DMA,
            pltpu.SemaphoreType.DMA,
        ],
        compiler_params=pltpu.CompilerParams(needs_layout_passes=False),
    )(_regulate_body)

    x_out = regulate(table, dur, jnp.zeros((_CH, _D), jnp.float32),
                     jnp.zeros((_M,), jnp.int32))
    return (x_out, log_pred)
